# baseline (speedup 1.0000x reference)
import functools

import jax
import jax.numpy as jnp
from jax.experimental import pallas as pl
from jax.experimental.pallas import tpu as pltpu

_FEATS = (160, 160, 128, 128)


def _ll_kernel(sp_ref, y_ref, wct_ref, c_ref, out_ref, *, n_elem, inv_nets):
    g = pl.program_id(0)

    @pl.when(g == 0)
    def _init():
        out_ref[...] = jnp.zeros_like(out_ref)

    acc = jnp.zeros_like(out_ref)
    for j in range(y_ref.shape[0]):
        ya = y_ref[j]
        d = jnp.dot(wct_ref[...], ya,
                    preferred_element_type=jnp.float32)
        spa = sp_ref[:, y_ref.shape[0] * g + j, :]
        eidx = jax.lax.broadcasted_iota(jnp.int32, d.shape, 0)
        onehot = (spa == eidx).astype(jnp.float32)
        acc = acc + jnp.sum(onehot * ((d + c_ref[...]) * inv_nets), axis=0,
                            keepdims=True)
    out_ref[...] += acc


@jax.jit
def kernel(species, y, W, b, self_energies):
    B, A, NETS, F = y.shape
    N_ELEM = W.shape[1]
    KF = NETS * F

    feats = jnp.asarray(_FEATS[:N_ELEM], dtype=jnp.int32)
    fmask = (jnp.arange(F, dtype=jnp.int32)[None, :] < feats[:, None])
    Wm = W * fmask[None, :, :].astype(W.dtype)
    wct = Wm.transpose(1, 0, 2).reshape(N_ELEM, KF)
    c = (b.sum(axis=0) + self_energies * NETS)[:, None]

    yt = jnp.transpose(y, (1, 2, 3, 0)).reshape(A, KF, B)
    spt = jnp.transpose(species, (1, 0)).reshape(1, A, B)

    AB = 4
    out = pl.pallas_call(
        functools.partial(_ll_kernel, n_elem=N_ELEM, inv_nets=1.0 / NETS),
        grid=(A // AB,),
        in_specs=[
            pl.BlockSpec((1, A, B), lambda g: (0, 0, 0)),
            pl.BlockSpec((AB, KF, B), lambda g: (g, 0, 0)),
            pl.BlockSpec((N_ELEM, KF), lambda g: (0, 0)),
            pl.BlockSpec((N_ELEM, 1), lambda g: (0, 0)),
        ],
        out_specs=pl.BlockSpec((1, B), lambda g: (0, 0)),
        out_shape=jax.ShapeDtypeStruct((1, B), jnp.float32),
        compiler_params=pltpu.CompilerParams(
            dimension_semantics=("arbitrary",)),
    )(spt, yt, wct, c)

    return (species, out.reshape(B))


# speedup vs baseline: 1.0493x; 1.0493x over previous
import functools

import jax
import jax.numpy as jnp
from jax.experimental import pallas as pl
from jax.experimental.pallas import tpu as pltpu

_FEATS = (160, 160, 128, 128)


def _ll_kernel(sp_ref, y_ref, wct_ref, c_ref, out_ref, *, n_elem, inv_nets):
    g = pl.program_id(0)

    @pl.when(g == 0)
    def _init():
        out_ref[...] = jnp.zeros_like(out_ref)

    acc = jnp.zeros_like(out_ref)
    for j in range(y_ref.shape[0]):
        ya = y_ref[j]
        d = jnp.dot(wct_ref[...], ya,
                    preferred_element_type=jnp.float32)
        spa = sp_ref[:, y_ref.shape[0] * g + j, :]
        eidx = jax.lax.broadcasted_iota(jnp.int32, d.shape, 0)
        onehot = (spa == eidx).astype(jnp.float32)
        acc = acc + jnp.sum(onehot * ((d + c_ref[...]) * inv_nets), axis=0,
                            keepdims=True)
    out_ref[...] += acc


@jax.jit
def kernel(species, y, W, b, self_energies):
    B, A, NETS, F = y.shape
    N_ELEM = W.shape[1]
    KF = NETS * F

    feats = jnp.asarray(_FEATS[:N_ELEM], dtype=jnp.int32)
    fmask = (jnp.arange(F, dtype=jnp.int32)[None, :] < feats[:, None])
    Wm = W * fmask[None, :, :].astype(W.dtype)
    wct = Wm.transpose(1, 0, 2).reshape(N_ELEM, KF)
    c = (b.sum(axis=0) + self_energies * NETS)[:, None]

    yt = jnp.transpose(y, (1, 2, 3, 0)).reshape(A, KF, B)
    spt = jnp.transpose(species, (1, 0)).reshape(1, A, B)

    AB = 2
    out = pl.pallas_call(
        functools.partial(_ll_kernel, n_elem=N_ELEM, inv_nets=1.0 / NETS),
        grid=(A // AB,),
        in_specs=[
            pl.BlockSpec((1, A, B), lambda g: (0, 0, 0)),
            pl.BlockSpec((AB, KF, B), lambda g: (g, 0, 0)),
            pl.BlockSpec((N_ELEM, KF), lambda g: (0, 0)),
            pl.BlockSpec((N_ELEM, 1), lambda g: (0, 0)),
        ],
        out_specs=pl.BlockSpec((1, B), lambda g: (0, 0)),
        out_shape=jax.ShapeDtypeStruct((1, B), jnp.float32),
        compiler_params=pltpu.CompilerParams(
            dimension_semantics=("arbitrary",)),
    )(spt, yt, wct, c)

    return (species, out.reshape(B))


# speedup vs baseline: 1.0639x; 1.0138x over previous
import functools

import jax
import jax.numpy as jnp
from jax import lax
from jax.experimental import pallas as pl
from jax.experimental.pallas import tpu as pltpu

_FEATS = (160, 160, 128, 128)


def _ll_kernel(sp_ref, y_ref, w_ref, b_ref, se_ref, out_ref, wct_s, c_s, *,
               n_elem, inv_nets):
    g = pl.program_id(0)
    n_nets, _, f_dim = w_ref.shape

    @pl.when(g == 0)
    def _prep():
        lane = lax.broadcasted_iota(jnp.int32, (n_elem, f_dim), 1)
        row = lax.broadcasted_iota(jnp.int32, (n_elem, f_dim), 0)
        fmask = jnp.zeros((n_elem, f_dim), jnp.float32)
        for e in range(n_elem):
            fmask = jnp.where((row == e) & (lane < _FEATS[e]), 1.0, fmask)
        for i in range(n_nets):
            wct_s[:, i * f_dim:(i + 1) * f_dim] = w_ref[i] * fmask
        ones = jnp.ones((1, n_nets), jnp.float32)
        c_s[...] = (jnp.dot(ones, b_ref[...], preferred_element_type=jnp.float32)
                    + se_ref[...] * n_nets)
        out_ref[...] = jnp.zeros_like(out_ref)

    acc = jnp.zeros_like(out_ref)
    for j in range(y_ref.shape[0]):
        ya = y_ref[j]
        d = jnp.dot(wct_s[...], ya,
                    preferred_element_type=jnp.float32)
        spa = sp_ref[:, y_ref.shape[0] * g + j, :]
        eidx = lax.broadcasted_iota(jnp.int32, d.shape, 0)
        onehot = (spa == eidx).astype(jnp.float32)
        acc = acc + (jnp.sum(onehot * d, axis=0, keepdims=True)
                     + jnp.dot(c_s[...], onehot,
                               preferred_element_type=jnp.float32))
    out_ref[...] += acc * inv_nets


@jax.jit
def kernel(species, y, W, b, self_energies):
    B, A, NETS, F = y.shape
    N_ELEM = W.shape[1]
    KF = NETS * F

    yt = jnp.transpose(y, (1, 2, 3, 0)).reshape(A, KF, B)
    spt = jnp.transpose(species, (1, 0)).reshape(1, A, B)
    se = self_energies.reshape(1, N_ELEM)

    AB = 2
    out = pl.pallas_call(
        functools.partial(_ll_kernel, n_elem=N_ELEM, inv_nets=1.0 / NETS),
        grid=(A // AB,),
        in_specs=[
            pl.BlockSpec((1, A, B), lambda g: (0, 0, 0)),
            pl.BlockSpec((AB, KF, B), lambda g: (g, 0, 0)),
            pl.BlockSpec((NETS, N_ELEM, F), lambda g: (0, 0, 0)),
            pl.BlockSpec((NETS, N_ELEM), lambda g: (0, 0)),
            pl.BlockSpec((1, N_ELEM), lambda g: (0, 0)),
        ],
        out_specs=pl.BlockSpec((1, B), lambda g: (0, 0)),
        out_shape=jax.ShapeDtypeStruct((1, B), jnp.float32),
        scratch_shapes=[
            pltpu.VMEM((N_ELEM, KF), jnp.float32),
            pltpu.VMEM((1, N_ELEM), jnp.float32),
        ],
        compiler_params=pltpu.CompilerParams(
            dimension_semantics=("arbitrary",)),
    )(spt, yt, W, b, se)

    return (species, out.reshape(B))


# speedup vs baseline: 1.1111x; 1.0444x over previous
import functools

import jax
import jax.numpy as jnp
from jax import lax
from jax.experimental import pallas as pl
from jax.experimental.pallas import tpu as pltpu

_FEATS = (160, 160, 128, 128)


def _ll_kernel(sp_ref, y_ref, w_ref, b_ref, se_ref, out_ref, wct_s, c_s, *,
               n_elem, inv_nets):
    g = pl.program_id(0)
    n_nets, _, f_dim = w_ref.shape

    @pl.when(g == 0)
    def _prep():
        lane = lax.broadcasted_iota(jnp.int32, (n_elem, f_dim), 1)
        row = lax.broadcasted_iota(jnp.int32, (n_elem, f_dim), 0)
        fmask = jnp.zeros((n_elem, f_dim), jnp.float32)
        for e in range(n_elem):
            fmask = jnp.where((row == e) & (lane < _FEATS[e]), 1.0, fmask)
        for i in range(n_nets):
            wct_s[:, i * f_dim:(i + 1) * f_dim] = w_ref[i] * fmask
        ones = jnp.ones((1, n_nets), jnp.float32)
        c_s[...] = (jnp.dot(ones, b_ref[...], preferred_element_type=jnp.float32,
                            precision=lax.Precision.HIGHEST)
                    + se_ref[...] * n_nets)
        out_ref[...] = jnp.zeros_like(out_ref)

    acc = jnp.zeros_like(out_ref)
    for j in range(y_ref.shape[0]):
        ya = y_ref[j]
        d = jnp.dot(wct_s[...], ya,
                    preferred_element_type=jnp.float32)
        spa = sp_ref[:, y_ref.shape[0] * g + j, :]
        eidx = lax.broadcasted_iota(jnp.int32, d.shape, 0)
        onehot = (spa == eidx).astype(jnp.float32)
        acc = acc + (jnp.sum(onehot * d, axis=0, keepdims=True)
                     + jnp.dot(c_s[...], onehot,
                               preferred_element_type=jnp.float32,
                               precision=lax.Precision.HIGHEST))
    out_ref[...] += acc * inv_nets


@jax.jit
def kernel(species, y, W, b, self_energies):
    B, A, NETS, F = y.shape
    N_ELEM = W.shape[1]
    KF = NETS * F

    yt = jnp.transpose(y, (1, 2, 3, 0)).reshape(A, KF, B)
    spt = jnp.transpose(species, (1, 0)).reshape(1, A, B)
    se = self_energies.reshape(1, N_ELEM)

    AB = 2
    out = pl.pallas_call(
        functools.partial(_ll_kernel, n_elem=N_ELEM, inv_nets=1.0 / NETS),
        grid=(A // AB,),
        in_specs=[
            pl.BlockSpec((1, A, B), lambda g: (0, 0, 0)),
            pl.BlockSpec((AB, KF, B), lambda g: (g, 0, 0)),
            pl.BlockSpec((NETS, N_ELEM, F), lambda g: (0, 0, 0)),
            pl.BlockSpec((NETS, N_ELEM), lambda g: (0, 0)),
            pl.BlockSpec((1, N_ELEM), lambda g: (0, 0)),
        ],
        out_specs=pl.BlockSpec((1, B), lambda g: (0, 0)),
        out_shape=jax.ShapeDtypeStruct((1, B), jnp.float32),
        scratch_shapes=[
            pltpu.VMEM((N_ELEM, KF), jnp.float32),
            pltpu.VMEM((1, N_ELEM), jnp.float32),
        ],
        compiler_params=pltpu.CompilerParams(
            dimension_semantics=("arbitrary",)),
    )(spt, yt, W, b, se)

    return (species, out.reshape(B))
